# baseline (speedup 1.0000x reference)
import functools
import math

import jax
import jax.numpy as jnp
import numpy as np
from jax.experimental import pallas as pl
from jax.experimental.pallas import tpu as pltpu

N = 10000
K = 28
NF = 64
NG = 25
NI = 4
CUTOFF = 6.0
NMOL = 128

NPAD = 10240
RA = 128
WA = 512
RB = 128
WB = 512
RC = 1024
KP = 32
TT = 384
EWMAX = 12.0
LOG2 = math.log(2.0)
PAD_MOL = 200.0


def _ssp(x):
    return jnp.maximum(x, 0.0) + jnp.log1p(jnp.exp(-jnp.abs(x))) - LOG2


def _knn_body(ws_ref, col_ref, row_ref, ewt_ref, nbrt_ref):
    c = pl.program_id(0)
    ws = ws_ref[c]
    win = col_ref[pl.ds(ws, WA), :]
    rowb = row_ref[...]
    d2 = (win[:, 3:4] + rowb[3:4, :]
          - 2.0 * jnp.dot(win[:, 0:3], rowb[0:3, :],
                          preferred_element_type=jnp.float32))
    gcol = win[:, 5:6]
    grow = rowb[5:6, :]
    valid = (win[:, 4:5] == rowb[4:5, :]) & (gcol != grow)
    d2 = jnp.where(valid, d2, jnp.inf)
    for k in range(K):
        m = jnp.minimum(jnp.min(d2, axis=0, keepdims=True), 1e9)
        idx = jnp.min(jnp.where(d2 == m, gcol, jnp.float32(2**30)),
                      axis=0, keepdims=True)
        ewt_ref[k:k + 1, :] = jnp.sqrt(jnp.maximum(m, 0.0) + 1e-12)
        nbrt_ref[k:k + 1, :] = idx
        d2 = jnp.where(gcol == idx, jnp.inf, d2)


def _onehot(idx_col, nlanes):
    lane = jax.lax.broadcasted_iota(jnp.int32, (1, nlanes), 1).astype(jnp.float32)
    return jnp.maximum(1.0 - jnp.abs(idx_col - lane), 0.0)


def _filtertab_body(w1_ref, b1_ref, w2_ref, b2_ref, tab_ref, *, step, coeff):
    dt = EWMAX / (TT - 1)
    ewg = jax.lax.broadcasted_iota(jnp.int32, (TT, 1), 0).astype(jnp.float32)
    ewg = ewg * dt
    klane = jax.lax.broadcasted_iota(jnp.int32, (1, KP), 1)
    offv = jnp.where(klane < NG, klane.astype(jnp.float32) * step, 1e9)
    ea = jnp.exp(coeff * (ewg - offv) ** 2)
    t = _ssp(jnp.dot(ea, w1_ref[0],
                     preferred_element_type=jnp.float32) + b1_ref[0])
    wf = jnp.dot(t, w2_ref[0], preferred_element_type=jnp.float32) + b2_ref[0]
    ck = 0.5 * (jnp.cos(ewg * (math.pi / CUTOFF)) + 1.0)
    tab_ref[0] = wf * ck


def _interact(src_ref, ws, c, ew_ref, nbr_ref, tab_ref, cw1_ref, cw2_ref,
              cb2_ref, wi_ref, bi_ref):
    hwin = src_ref[pl.ds(ws, WB), :]
    xlwin = jnp.dot(hwin, cw1_ref[0],
                    preferred_element_type=jnp.float32)
    lane = jax.lax.broadcasted_iota(jnp.int32, (1, WB), 1).astype(jnp.float32)
    gcolw = ws.astype(jnp.float32) + lane
    tlane = jax.lax.broadcasted_iota(jnp.int32, (1, TT), 1).astype(jnp.float32)
    invdt = (TT - 1) / EWMAX
    acc = jnp.zeros((RB, NF), dtype=jnp.float32)
    for k in range(K):
        ewk = ew_ref[:, k:k + 1]
        p = ewk * invdt
        lerp = jnp.maximum(1.0 - jnp.abs(p - tlane), 0.0)
        wf = jnp.dot(lerp, tab_ref[0], preferred_element_type=jnp.float32)
        nk = nbr_ref[:, k:k + 1]
        sel = jnp.maximum(1.0 - jnp.abs(nk - gcolw), 0.0)
        g = jnp.dot(sel, xlwin, preferred_element_type=jnp.float32)
        acc = acc + wf * g
    v = jnp.dot(acc, cw2_ref[0],
                preferred_element_type=jnp.float32) + cb2_ref[0]
    v = _ssp(v)
    v = jnp.dot(v, wi_ref[0], preferred_element_type=jnp.float32) + bi_ref[0]
    return src_ref[pl.ds(c * RB, RB), :] + v


def _mega_body(ws_ref, zf_ref, bf_ref, ew_ref, nbr_ref, emb_ref, tab_ref,
               cw1_ref, cw2_ref, cb2_ref, wi_ref, bi_ref, l1_ref, l1b_ref,
               l2_ref, l2b_ref, out_ref, ha_ref, hb_ref):
    i = pl.program_id(0)
    c = pl.program_id(1)
    ws = ws_ref[c]

    @pl.when(i == 0)
    def _():
        onehot = _onehot(zf_ref[...], 128)
        ha_ref[pl.ds(c * RB, RB), :] = jnp.dot(
            onehot, emb_ref[...], preferred_element_type=jnp.float32)

    for ph in range(1, NI + 1):
        src_ref, dst_ref = (ha_ref, hb_ref) if ph % 2 == 1 else (hb_ref, ha_ref)

        @pl.when(i == ph)
        def _(src_ref=src_ref, dst_ref=dst_ref, ph=ph):
            hnew = _interact(src_ref, ws, c, ew_ref, nbr_ref, tab_ref,
                             cw1_ref, cw2_ref, cb2_ref, wi_ref, bi_ref)
            dst_ref[pl.ds(c * RB, RB), :] = hnew
            if ph == NI:
                t = _ssp(jnp.dot(hnew, l1_ref[...],
                                 preferred_element_type=jnp.float32)
                         + l1b_ref[...])
                y = (jnp.dot(t, l2_ref[...],
                             preferred_element_type=jnp.float32)
                     + l2b_ref[...])
                y = jnp.where(zf_ref[...] == 0.0, 0.0, y)
                oneb = _onehot(bf_ref[...], NMOL)
                part = jax.lax.dot_general(
                    y, oneb, (((0,), (0,)), ((), ())),
                    preferred_element_type=jnp.float32)

                @pl.when(c == 0)
                def _():
                    out_ref[...] = part

                @pl.when(c > 0)
                def _():
                    out_ref[...] += part


def kernel(z, pos, batch, emb, mlp_w1, mlp_b1, mlp_w2, mlp_b2, conv_w1,
           conv_w2, conv_b2, int_lin_w, int_lin_b, lin1_w, lin1_b, lin2_w,
           lin2_b):
    f32 = jnp.float32
    npad = NPAD - N
    posp = jnp.pad(pos.astype(f32), ((0, npad), (0, 0)))
    sq = jnp.sum(posp * posp, axis=1)
    batch_i = jnp.pad(batch.astype(jnp.int32), (0, npad), constant_values=1000)
    batchf = jnp.pad(batch.astype(f32), (0, npad), constant_values=PAD_MOL)
    zf = jnp.pad(z.astype(f32), (0, npad))
    gidx = jnp.arange(NPAD, dtype=f32)
    zero = jnp.zeros((NPAD,), dtype=f32)
    colpack = jnp.stack([posp[:, 0], posp[:, 1], posp[:, 2], sq, batchf,
                         gidx, zero, zero], axis=1)
    rowpack = colpack.T

    qa = batch_i[jnp.arange(NPAD // RA) * RA]
    ws_a = jnp.searchsorted(batch_i, qa).astype(jnp.int32)
    ws_a = jnp.minimum((ws_a // 8) * 8, NPAD - WA)
    qb = batch_i[jnp.arange(NPAD // RB) * RB]
    ws_b = jnp.searchsorted(batch_i, qb).astype(jnp.int32)
    ws_b = jnp.minimum((ws_b // 8) * 8, NPAD - WB)

    grid_a = pltpu.PrefetchScalarGridSpec(
        num_scalar_prefetch=1,
        grid=(NPAD // RA,),
        in_specs=[
            pl.BlockSpec((NPAD, 8), lambda c, ws: (0, 0)),
            pl.BlockSpec((8, RA), lambda c, ws: (0, c)),
        ],
        out_specs=[
            pl.BlockSpec((KP, RA), lambda c, ws: (0, c)),
            pl.BlockSpec((KP, RA), lambda c, ws: (0, c)),
        ],
    )
    ewt, nbrt = pl.pallas_call(
        _knn_body,
        grid_spec=grid_a,
        out_shape=[
            jax.ShapeDtypeStruct((KP, NPAD), f32),
            jax.ShapeDtypeStruct((KP, NPAD), f32),
        ],
    )(ws_a, colpack, rowpack)
    ew = ewt.T
    nbr = nbrt.T

    step = float(CUTOFF / (NG - 1))
    coeff = float(-0.5 / step ** 2)
    w1p = jnp.pad(mlp_w1.astype(f32), ((0, 0), (0, KP - NG), (0, 0)))
    tabs = pl.pallas_call(
        functools.partial(_filtertab_body, step=step, coeff=coeff),
        grid=(NI,),
        in_specs=[
            pl.BlockSpec((1, KP, NF), lambda i: (i, 0, 0)),
            pl.BlockSpec((1, 1, NF), lambda i: (i, 0, 0)),
            pl.BlockSpec((1, NF, NF), lambda i: (i, 0, 0)),
            pl.BlockSpec((1, 1, NF), lambda i: (i, 0, 0)),
        ],
        out_specs=pl.BlockSpec((1, TT, NF), lambda i: (i, 0, 0)),
        out_shape=jax.ShapeDtypeStruct((NI, TT, NF), f32),
    )(w1p, mlp_b1.astype(f32)[:, None, :], mlp_w2.astype(f32),
      mlp_b2.astype(f32)[:, None, :])
    emb_pad = jnp.pad(emb.astype(f32), ((0, 128 - emb.shape[0]), (0, 0)))
    wsel = lambda i, c, ws: (jnp.maximum(i - 1, 0), 0, 0)
    grid_m = pltpu.PrefetchScalarGridSpec(
        num_scalar_prefetch=1,
        grid=(NI + 1, NPAD // RB),
        in_specs=[
            pl.BlockSpec((RB, 1), lambda i, c, ws: (c, 0)),
            pl.BlockSpec((RB, 1), lambda i, c, ws: (c, 0)),
            pl.BlockSpec((RB, KP), lambda i, c, ws: (c, 0)),
            pl.BlockSpec((RB, KP), lambda i, c, ws: (c, 0)),
            pl.BlockSpec((128, NF), lambda i, c, ws: (0, 0)),
            pl.BlockSpec((1, TT, NF), wsel),
            pl.BlockSpec((1, NF, NF), wsel),
            pl.BlockSpec((1, NF, NF), wsel),
            pl.BlockSpec((1, 1, NF), wsel),
            pl.BlockSpec((1, NF, NF), wsel),
            pl.BlockSpec((1, 1, NF), wsel),
            pl.BlockSpec((NF, NF // 2), lambda i, c, ws: (0, 0)),
            pl.BlockSpec((1, NF // 2), lambda i, c, ws: (0, 0)),
            pl.BlockSpec((NF // 2, 1), lambda i, c, ws: (0, 0)),
            pl.BlockSpec((1, 1), lambda i, c, ws: (0, 0)),
        ],
        out_specs=pl.BlockSpec((1, NMOL), lambda i, c, ws: (0, 0)),
        scratch_shapes=[
            pltpu.VMEM((NPAD, NF), f32),
            pltpu.VMEM((NPAD, NF), f32),
        ],
    )
    out = pl.pallas_call(
        _mega_body,
        grid_spec=grid_m,
        out_shape=jax.ShapeDtypeStruct((1, NMOL), f32),
    )(ws_b, zf[:, None], batchf[:, None], ew, nbr, emb_pad, tabs,
      conv_w1.astype(f32), conv_w2.astype(f32),
      conv_b2.astype(f32)[:, None, :], int_lin_w.astype(f32),
      int_lin_b.astype(f32)[:, None, :], lin1_w.astype(f32),
      lin1_b.astype(f32)[None, :], lin2_w.astype(f32),
      lin2_b.astype(f32)[None, :])
    return out.reshape(-1)


# speedup vs baseline: 1.0814x; 1.0814x over previous
import functools
import math

import jax
import jax.numpy as jnp
import numpy as np
from jax.experimental import pallas as pl
from jax.experimental.pallas import tpu as pltpu

N = 10000
K = 28
NF = 64
NG = 25
NI = 4
CUTOFF = 6.0
NMOL = 128

NPAD = 10240
RA = 128
WA = 512
RB = 256
WB = 640
RC = 1024
KP = 32
TT = 384
EWMAX = 12.0
LOG2 = math.log(2.0)
PAD_MOL = 200.0


def _ssp(x):
    return jnp.maximum(x, 0.0) + jnp.log1p(jnp.exp(-jnp.abs(x))) - LOG2


def _knn_body(ws_ref, col_ref, row_ref, ewt_ref, nbrt_ref):
    c = pl.program_id(0)
    ws = ws_ref[c]
    win = col_ref[pl.ds(ws, WA), :]
    rowb = row_ref[...]
    d2 = (win[:, 3:4] + rowb[3:4, :]
          - 2.0 * jnp.dot(win[:, 0:3], rowb[0:3, :],
                          preferred_element_type=jnp.float32))
    gcol = win[:, 5:6]
    grow = rowb[5:6, :]
    valid = (win[:, 4:5] == rowb[4:5, :]) & (gcol != grow)
    d2 = jnp.where(valid, d2, jnp.inf)
    for k in range(K):
        m = jnp.minimum(jnp.min(d2, axis=0, keepdims=True), 1e9)
        idx = jnp.min(jnp.where(d2 == m, gcol, jnp.float32(2**30)),
                      axis=0, keepdims=True)
        ewt_ref[k:k + 1, :] = jnp.sqrt(jnp.maximum(m, 0.0) + 1e-12)
        nbrt_ref[k:k + 1, :] = idx
        d2 = jnp.where(gcol == idx, jnp.inf, d2)


def _onehot(idx_col, nlanes):
    lane = jax.lax.broadcasted_iota(jnp.int32, (1, nlanes), 1).astype(jnp.float32)
    return jnp.maximum(1.0 - jnp.abs(idx_col - lane), 0.0)


def _filtertab_body(w1_ref, b1_ref, w2_ref, b2_ref, tab_ref, *, step, coeff):
    dt = EWMAX / (TT - 1)
    ewg = jax.lax.broadcasted_iota(jnp.int32, (TT, 1), 0).astype(jnp.float32)
    ewg = ewg * dt
    klane = jax.lax.broadcasted_iota(jnp.int32, (1, KP), 1)
    offv = jnp.where(klane < NG, klane.astype(jnp.float32) * step, 1e9)
    ea = jnp.exp(coeff * (ewg - offv) ** 2)
    t = _ssp(jnp.dot(ea, w1_ref[0],
                     preferred_element_type=jnp.float32) + b1_ref[0])
    wf = jnp.dot(t, w2_ref[0], preferred_element_type=jnp.float32) + b2_ref[0]
    ck = 0.5 * (jnp.cos(ewg * (math.pi / CUTOFF)) + 1.0)
    tab_ref[0] = wf * ck


def _interact(src_ref, ws, c, ew_ref, nbr_ref, tab_ref, cw1_ref, cw2_ref,
              cb2_ref, wi_ref, bi_ref):
    hwin = src_ref[pl.ds(ws, WB), :]
    xlwin = jnp.dot(hwin, cw1_ref[0],
                    preferred_element_type=jnp.float32)
    lane = jax.lax.broadcasted_iota(jnp.int32, (1, WB), 1).astype(jnp.float32)
    gcolw = ws.astype(jnp.float32) + lane
    tlane = jax.lax.broadcasted_iota(jnp.int32, (1, TT), 1).astype(jnp.float32)
    invdt = (TT - 1) / EWMAX
    acc = jnp.zeros((RB, NF), dtype=jnp.float32)
    for k in range(K):
        ewk = ew_ref[:, k:k + 1]
        p = ewk * invdt
        lerp = jnp.maximum(1.0 - jnp.abs(p - tlane), 0.0)
        wf = jnp.dot(lerp, tab_ref[0], preferred_element_type=jnp.float32)
        nk = nbr_ref[:, k:k + 1]
        sel = jnp.maximum(1.0 - jnp.abs(nk - gcolw), 0.0)
        g = jnp.dot(sel, xlwin, preferred_element_type=jnp.float32)
        acc = acc + wf * g
    v = jnp.dot(acc, cw2_ref[0],
                preferred_element_type=jnp.float32) + cb2_ref[0]
    v = _ssp(v)
    v = jnp.dot(v, wi_ref[0], preferred_element_type=jnp.float32) + bi_ref[0]
    return src_ref[pl.ds(c * RB, RB), :] + v


def _mega_body(ws_ref, zf_ref, bf_ref, ew_ref, nbr_ref, emb_ref, tab_ref,
               cw1_ref, cw2_ref, cb2_ref, wi_ref, bi_ref, l1_ref, l1b_ref,
               l2_ref, l2b_ref, out_ref, ha_ref, hb_ref):
    i = pl.program_id(0)
    c = pl.program_id(1)
    ws = ws_ref[c]

    @pl.when(i == 0)
    def _():
        onehot = _onehot(zf_ref[...], 128)
        ha_ref[pl.ds(c * RB, RB), :] = jnp.dot(
            onehot, emb_ref[...], preferred_element_type=jnp.float32)

    for ph in range(1, NI + 1):
        src_ref, dst_ref = (ha_ref, hb_ref) if ph % 2 == 1 else (hb_ref, ha_ref)

        @pl.when(i == ph)
        def _(src_ref=src_ref, dst_ref=dst_ref, ph=ph):
            hnew = _interact(src_ref, ws, c, ew_ref, nbr_ref, tab_ref,
                             cw1_ref, cw2_ref, cb2_ref, wi_ref, bi_ref)
            dst_ref[pl.ds(c * RB, RB), :] = hnew
            if ph == NI:
                t = _ssp(jnp.dot(hnew, l1_ref[...],
                                 preferred_element_type=jnp.float32)
                         + l1b_ref[...])
                y = (jnp.dot(t, l2_ref[...],
                             preferred_element_type=jnp.float32)
                     + l2b_ref[...])
                y = jnp.where(zf_ref[...] == 0.0, 0.0, y)
                oneb = _onehot(bf_ref[...], NMOL)
                part = jax.lax.dot_general(
                    y, oneb, (((0,), (0,)), ((), ())),
                    preferred_element_type=jnp.float32)

                @pl.when(c == 0)
                def _():
                    out_ref[...] = part

                @pl.when(c > 0)
                def _():
                    out_ref[...] += part


def kernel(z, pos, batch, emb, mlp_w1, mlp_b1, mlp_w2, mlp_b2, conv_w1,
           conv_w2, conv_b2, int_lin_w, int_lin_b, lin1_w, lin1_b, lin2_w,
           lin2_b):
    f32 = jnp.float32
    npad = NPAD - N
    posp = jnp.pad(pos.astype(f32), ((0, npad), (0, 0)))
    sq = jnp.sum(posp * posp, axis=1)
    batch_i = jnp.pad(batch.astype(jnp.int32), (0, npad), constant_values=1000)
    batchf = jnp.pad(batch.astype(f32), (0, npad), constant_values=PAD_MOL)
    zf = jnp.pad(z.astype(f32), (0, npad))
    gidx = jnp.arange(NPAD, dtype=f32)
    zero = jnp.zeros((NPAD,), dtype=f32)
    colpack = jnp.stack([posp[:, 0], posp[:, 1], posp[:, 2], sq, batchf,
                         gidx, zero, zero], axis=1)
    rowpack = colpack.T

    qa = batch_i[jnp.arange(NPAD // RA) * RA]
    ws_a = jnp.searchsorted(batch_i, qa).astype(jnp.int32)
    ws_a = jnp.minimum((ws_a // 8) * 8, NPAD - WA)
    qb = batch_i[jnp.arange(NPAD // RB) * RB]
    ws_b = jnp.searchsorted(batch_i, qb).astype(jnp.int32)
    ws_b = jnp.minimum((ws_b // 8) * 8, NPAD - WB)

    grid_a = pltpu.PrefetchScalarGridSpec(
        num_scalar_prefetch=1,
        grid=(NPAD // RA,),
        in_specs=[
            pl.BlockSpec((NPAD, 8), lambda c, ws: (0, 0)),
            pl.BlockSpec((8, RA), lambda c, ws: (0, c)),
        ],
        out_specs=[
            pl.BlockSpec((KP, RA), lambda c, ws: (0, c)),
            pl.BlockSpec((KP, RA), lambda c, ws: (0, c)),
        ],
    )
    ewt, nbrt = pl.pallas_call(
        _knn_body,
        grid_spec=grid_a,
        out_shape=[
            jax.ShapeDtypeStruct((KP, NPAD), f32),
            jax.ShapeDtypeStruct((KP, NPAD), f32),
        ],
    )(ws_a, colpack, rowpack)
    ew = ewt.T
    nbr = nbrt.T

    step = float(CUTOFF / (NG - 1))
    coeff = float(-0.5 / step ** 2)
    w1p = jnp.pad(mlp_w1.astype(f32), ((0, 0), (0, KP - NG), (0, 0)))
    tabs = pl.pallas_call(
        functools.partial(_filtertab_body, step=step, coeff=coeff),
        grid=(NI,),
        in_specs=[
            pl.BlockSpec((1, KP, NF), lambda i: (i, 0, 0)),
            pl.BlockSpec((1, 1, NF), lambda i: (i, 0, 0)),
            pl.BlockSpec((1, NF, NF), lambda i: (i, 0, 0)),
            pl.BlockSpec((1, 1, NF), lambda i: (i, 0, 0)),
        ],
        out_specs=pl.BlockSpec((1, TT, NF), lambda i: (i, 0, 0)),
        out_shape=jax.ShapeDtypeStruct((NI, TT, NF), f32),
    )(w1p, mlp_b1.astype(f32)[:, None, :], mlp_w2.astype(f32),
      mlp_b2.astype(f32)[:, None, :])
    emb_pad = jnp.pad(emb.astype(f32), ((0, 128 - emb.shape[0]), (0, 0)))
    wsel = lambda i, c, ws: (jnp.maximum(i - 1, 0), 0, 0)
    grid_m = pltpu.PrefetchScalarGridSpec(
        num_scalar_prefetch=1,
        grid=(NI + 1, NPAD // RB),
        in_specs=[
            pl.BlockSpec((RB, 1), lambda i, c, ws: (c, 0)),
            pl.BlockSpec((RB, 1), lambda i, c, ws: (c, 0)),
            pl.BlockSpec((RB, KP), lambda i, c, ws: (c, 0)),
            pl.BlockSpec((RB, KP), lambda i, c, ws: (c, 0)),
            pl.BlockSpec((128, NF), lambda i, c, ws: (0, 0)),
            pl.BlockSpec((1, TT, NF), wsel),
            pl.BlockSpec((1, NF, NF), wsel),
            pl.BlockSpec((1, NF, NF), wsel),
            pl.BlockSpec((1, 1, NF), wsel),
            pl.BlockSpec((1, NF, NF), wsel),
            pl.BlockSpec((1, 1, NF), wsel),
            pl.BlockSpec((NF, NF // 2), lambda i, c, ws: (0, 0)),
            pl.BlockSpec((1, NF // 2), lambda i, c, ws: (0, 0)),
            pl.BlockSpec((NF // 2, 1), lambda i, c, ws: (0, 0)),
            pl.BlockSpec((1, 1), lambda i, c, ws: (0, 0)),
        ],
        out_specs=pl.BlockSpec((1, NMOL), lambda i, c, ws: (0, 0)),
        scratch_shapes=[
            pltpu.VMEM((NPAD, NF), f32),
            pltpu.VMEM((NPAD, NF), f32),
        ],
    )
    out = pl.pallas_call(
        _mega_body,
        grid_spec=grid_m,
        out_shape=jax.ShapeDtypeStruct((1, NMOL), f32),
    )(ws_b, zf[:, None], batchf[:, None], ew, nbr, emb_pad, tabs,
      conv_w1.astype(f32), conv_w2.astype(f32),
      conv_b2.astype(f32)[:, None, :], int_lin_w.astype(f32),
      int_lin_b.astype(f32)[:, None, :], lin1_w.astype(f32),
      lin1_b.astype(f32)[None, :], lin2_w.astype(f32),
      lin2_b.astype(f32)[None, :])
    return out.reshape(-1)
